# Initial kernel scaffold; baseline (speedup 1.0000x reference)
import jax
import jax.numpy as jnp
from jax.experimental import pallas as pl


def kernel(edge_index, edge_attr):
    raise NotImplementedError("write your pallas kernel here")


# speedup vs baseline: 2.1239x; 2.1239x over previous
import functools

import numpy as np
import jax
import jax.numpy as jnp
from jax import lax
from jax.experimental import pallas as pl
from jax.experimental.pallas import tpu as pltpu
from jax.experimental.pallas import tpu_sc as plsc

_NC = 2
_NS = 16
_NW = _NC * _NS

_keep_cache = {}


def _threefry_block(k0, k1, x0, x1):
    rot_a = (13, 15, 26, 6)
    rot_b = (17, 29, 16, 24)

    def rotl(x, r):
        r = np.uint32(r)
        return (x << r) | (x >> np.uint32(32 - r))

    with np.errstate(over="ignore"):
        ks0 = np.uint32(k0)
        ks1 = np.uint32(k1)
        ks2 = np.uint32(ks0 ^ ks1 ^ np.uint32(0x1BD11BDA))
        x0 = x0.astype(np.uint32) + ks0
        x1 = x1.astype(np.uint32) + ks1

        def round4(x0, x1, rots):
            for r in rots:
                x0 = x0 + x1
                x1 = rotl(x1, r)
                x1 = x1 ^ x0
            return x0, x1

        x0, x1 = round4(x0, x1, rot_a)
        x0 = x0 + ks1; x1 = x1 + ks2 + np.uint32(1)
        x0, x1 = round4(x0, x1, rot_b)
        x0 = x0 + ks2; x1 = x1 + ks0 + np.uint32(2)
        x0, x1 = round4(x0, x1, rot_a)
        x0 = x0 + ks0; x1 = x1 + ks1 + np.uint32(3)
        x0, x1 = round4(x0, x1, rot_b)
        x0 = x0 + ks1; x1 = x1 + ks2 + np.uint32(4)
        x0, x1 = round4(x0, x1, rot_a)
        x0 = x0 + ks2; x1 = x1 + ks0 + np.uint32(5)
    return x0, x1


def _np_permutation(seed: int, n: int) -> np.ndarray:
    k0, k1 = np.uint32(seed >> 32), np.uint32(seed & 0xFFFFFFFF)
    x = np.arange(n, dtype=np.int32)
    iota_hi = np.zeros(n, dtype=np.uint32)
    iota_lo = np.arange(n, dtype=np.uint32)
    two_hi = np.zeros(2, dtype=np.uint32)
    two_lo = np.arange(2, dtype=np.uint32)
    num_rounds = int(np.ceil(3 * np.log(max(1, n)) / np.log(2**32 - 1)))
    for _ in range(num_rounds):
        y0, y1 = _threefry_block(k0, k1, two_hi, two_lo)
        (k0, k1), (s0, s1) = (y0[0], y1[0]), (y0[1], y1[1])
        b0, b1 = _threefry_block(s0, s1, iota_hi, iota_lo)
        x = x[np.argsort(b0 ^ b1, kind="stable")]
    return x


def _keep_indices(num_edges: int) -> np.ndarray:
    if num_edges not in _keep_cache:
        perm = _np_permutation(42, num_edges)
        num_drops = int(0.5 * num_edges)
        _keep_cache[num_edges] = perm[num_drops:].astype(np.int32)
    return _keep_cache[num_edges]


def _make_gather(E: int, K: int, D: int, chunk: int):
    per_w = K // _NW
    n_chunks = per_w // chunk
    mesh = plsc.VectorSubcoreMesh(core_axis_name="c", subcore_axis_name="s", num_cores=_NC, num_subcores=_NS)

    @functools.partial(
        pl.kernel,
        mesh=mesh,
        out_type=(
            jax.ShapeDtypeStruct((2 * K,), jnp.int32),
            jax.ShapeDtypeStruct((K * D,), jnp.float32),
        ),
        scratch_types=[
            pltpu.VMEM((chunk,), jnp.int32),
            pltpu.VMEM((chunk,), jnp.int32),
            pltpu.VMEM((chunk * 4,), jnp.int32),
            pltpu.VMEM((chunk,), jnp.int32),
            pltpu.VMEM((chunk,), jnp.int32),
            pltpu.VMEM((chunk * 4,), jnp.float32),
            pltpu.SemaphoreType.DMA,
        ],
        compiler_params=pltpu.CompilerParams(use_tc_tiling_on_sc=False),
    )
    def gather_kernel(ei_flat, attr_flat, keep_lo, keep_hi, keep4, out_ei,
                      out_attr, idx_lo_v, idx_hi_v, idx4_v, i0_v, i1_v,
                      attr_v, sem):
        wid = lax.axis_index("s") * _NC + lax.axis_index("c")
        w_base = wid * per_w

        def body(c, carry):
            base = w_base + c * chunk
            pltpu.sync_copy(keep_lo.at[pl.ds(base, chunk)], idx_lo_v)
            pltpu.sync_copy(keep_hi.at[pl.ds(base, chunk)], idx_hi_v)
            pltpu.sync_copy(keep4.at[pl.ds(4 * base, 4 * chunk)], idx4_v)
            cp_a = pltpu.async_copy(attr_flat.at[idx4_v], attr_v, sem)
            cp_0 = pltpu.async_copy(ei_flat.at[idx_lo_v], i0_v, sem)
            cp_1 = pltpu.async_copy(ei_flat.at[idx_hi_v], i1_v, sem)
            cp_a.wait()
            cp_0.wait()
            cp_1.wait()
            pltpu.sync_copy(attr_v, out_attr.at[pl.ds(4 * base, 4 * chunk)])
            pltpu.sync_copy(i0_v, out_ei.at[pl.ds(base, chunk)])
            pltpu.sync_copy(i1_v, out_ei.at[pl.ds(K + base, chunk)])
            return carry

        lax.fori_loop(0, n_chunks, body, 0)

    return gather_kernel


def kernel(edge_index, edge_attr):
    E = edge_index.shape[1]
    D = edge_attr.shape[1]
    K = E - int(0.5 * E)
    keep = _keep_indices(E)
    keep_lo = jnp.asarray(keep)
    keep_hi = jnp.asarray(keep + np.int32(E))
    keep4 = jnp.asarray(
        (keep.astype(np.int64)[:, None] * D
         + np.arange(D, dtype=np.int64)).astype(np.int32).reshape(-1))

    chunk = 5000
    assert K % (_NW * chunk) == 0

    ei_flat = edge_index.reshape(2 * E)
    attr_flat = edge_attr.reshape(E * D)
    gather_kernel = _make_gather(E, K, D, chunk)
    out_ei, out_attr = gather_kernel(ei_flat, attr_flat, keep_lo, keep_hi,
                                     keep4)
    return out_ei.reshape(2, K), out_attr.reshape(K, D)


# speedup vs baseline: 2.1447x; 1.0098x over previous
import functools

import numpy as np
import jax
import jax.numpy as jnp
from jax import lax
from jax.experimental import pallas as pl
from jax.experimental.pallas import tpu as pltpu
from jax.experimental.pallas import tpu_sc as plsc

_NC = 2
_NS = 16
_NW = _NC * _NS

_keep_cache = {}


def _threefry_block(k0, k1, x0, x1):
    rot_a = (13, 15, 26, 6)
    rot_b = (17, 29, 16, 24)

    def rotl(x, r):
        r = np.uint32(r)
        return (x << r) | (x >> np.uint32(32 - r))

    with np.errstate(over="ignore"):
        ks0 = np.uint32(k0)
        ks1 = np.uint32(k1)
        ks2 = np.uint32(ks0 ^ ks1 ^ np.uint32(0x1BD11BDA))
        x0 = x0.astype(np.uint32) + ks0
        x1 = x1.astype(np.uint32) + ks1

        def round4(x0, x1, rots):
            for r in rots:
                x0 = x0 + x1
                x1 = rotl(x1, r)
                x1 = x1 ^ x0
            return x0, x1

        x0, x1 = round4(x0, x1, rot_a)
        x0 = x0 + ks1; x1 = x1 + ks2 + np.uint32(1)
        x0, x1 = round4(x0, x1, rot_b)
        x0 = x0 + ks2; x1 = x1 + ks0 + np.uint32(2)
        x0, x1 = round4(x0, x1, rot_a)
        x0 = x0 + ks0; x1 = x1 + ks1 + np.uint32(3)
        x0, x1 = round4(x0, x1, rot_b)
        x0 = x0 + ks1; x1 = x1 + ks2 + np.uint32(4)
        x0, x1 = round4(x0, x1, rot_a)
        x0 = x0 + ks2; x1 = x1 + ks0 + np.uint32(5)
    return x0, x1


def _np_permutation(seed: int, n: int) -> np.ndarray:
    k0, k1 = np.uint32(seed >> 32), np.uint32(seed & 0xFFFFFFFF)
    x = np.arange(n, dtype=np.int32)
    iota_hi = np.zeros(n, dtype=np.uint32)
    iota_lo = np.arange(n, dtype=np.uint32)
    two_hi = np.zeros(2, dtype=np.uint32)
    two_lo = np.arange(2, dtype=np.uint32)
    num_rounds = int(np.ceil(3 * np.log(max(1, n)) / np.log(2**32 - 1)))
    for _ in range(num_rounds):
        y0, y1 = _threefry_block(k0, k1, two_hi, two_lo)
        (k0, k1), (s0, s1) = (y0[0], y1[0]), (y0[1], y1[1])
        b0, b1 = _threefry_block(s0, s1, iota_hi, iota_lo)
        x = x[np.argsort(b0 ^ b1, kind="stable")]
    return x


def _keep_indices(num_edges: int) -> np.ndarray:
    if num_edges not in _keep_cache:
        perm = _np_permutation(42, num_edges)
        num_drops = int(0.5 * num_edges)
        _keep_cache[num_edges] = perm[num_drops:].astype(np.int32)
    return _keep_cache[num_edges]


def _make_gather(E: int, K: int, D: int, chunk: int):
    per_w = K // _NW
    n_chunks = per_w // chunk
    mesh = plsc.VectorSubcoreMesh(core_axis_name="c", subcore_axis_name="s", num_cores=_NC, num_subcores=_NS)

    @functools.partial(
        pl.kernel,
        mesh=mesh,
        out_type=(
            jax.ShapeDtypeStruct((K,), jnp.int32),
            jax.ShapeDtypeStruct((K,), jnp.int32),
            jax.ShapeDtypeStruct((K * D,), jnp.float32),
        ),
        scratch_types=[
            pltpu.VMEM((chunk,), jnp.int32),
            pltpu.VMEM((chunk * 4,), jnp.int32),
            pltpu.VMEM((chunk,), jnp.int32),
            pltpu.VMEM((chunk,), jnp.int32),
            pltpu.VMEM((chunk * 4,), jnp.float32),
            pltpu.SemaphoreType.DMA,
        ],
        compiler_params=pltpu.CompilerParams(use_tc_tiling_on_sc=False),
    )
    def gather_kernel(ei0, ei1, attr_flat, keep_lo, keep4, out_i0, out_i1,
                      out_attr, idx_lo_v, idx4_v, i0_v, i1_v, attr_v, sem):
        wid = lax.axis_index("s") * _NC + lax.axis_index("c")
        w_base = wid * per_w

        def body(c, carry):
            base = w_base + c * chunk
            pltpu.sync_copy(keep_lo.at[pl.ds(base, chunk)], idx_lo_v)
            pltpu.sync_copy(keep4.at[pl.ds(4 * base, 4 * chunk)], idx4_v)
            cp_a = pltpu.async_copy(attr_flat.at[idx4_v], attr_v, sem)
            cp_0 = pltpu.async_copy(ei0.at[idx_lo_v], i0_v, sem)
            cp_1 = pltpu.async_copy(ei1.at[idx_lo_v], i1_v, sem)
            cp_a.wait()
            cp_0.wait()
            cp_1.wait()
            pltpu.sync_copy(attr_v, out_attr.at[pl.ds(4 * base, 4 * chunk)])
            pltpu.sync_copy(i0_v, out_i0.at[pl.ds(base, chunk)])
            pltpu.sync_copy(i1_v, out_i1.at[pl.ds(base, chunk)])
            return carry

        lax.fori_loop(0, n_chunks, body, 0)

    return gather_kernel


def kernel(edge_index, edge_attr):
    E = edge_index.shape[1]
    D = edge_attr.shape[1]
    K = E - int(0.5 * E)
    keep = _keep_indices(E)
    keep_lo = jnp.asarray(keep)
    keep4 = jnp.asarray(
        (keep.astype(np.int64)[:, None] * D
         + np.arange(D, dtype=np.int64)).astype(np.int32).reshape(-1))

    chunk = 5000
    assert K % (_NW * chunk) == 0

    attr_flat = edge_attr.reshape(E * D)
    gather_kernel = _make_gather(E, K, D, chunk)
    out_i0, out_i1, out_attr = gather_kernel(edge_index[0], edge_index[1],
                                             attr_flat, keep_lo, keep4)
    return jnp.stack([out_i0, out_i1]), out_attr.reshape(K, D)


# speedup vs baseline: 22.6479x; 10.5598x over previous
import functools

import numpy as np
import jax
import jax.numpy as jnp
from jax import lax
from jax.experimental import pallas as pl
from jax.experimental.pallas import tpu as pltpu
from jax.experimental.pallas import tpu_sc as plsc

_NC = 2
_NS = 16
_NW = _NC * _NS

_keep_cache = {}


def _threefry_block(k0, k1, x0, x1):
    rot_a = (13, 15, 26, 6)
    rot_b = (17, 29, 16, 24)

    def rotl(x, r):
        r = np.uint32(r)
        return (x << r) | (x >> np.uint32(32 - r))

    with np.errstate(over="ignore"):
        ks0 = np.uint32(k0)
        ks1 = np.uint32(k1)
        ks2 = np.uint32(ks0 ^ ks1 ^ np.uint32(0x1BD11BDA))
        x0 = x0.astype(np.uint32) + ks0
        x1 = x1.astype(np.uint32) + ks1

        def round4(x0, x1, rots):
            for r in rots:
                x0 = x0 + x1
                x1 = rotl(x1, r)
                x1 = x1 ^ x0
            return x0, x1

        x0, x1 = round4(x0, x1, rot_a)
        x0 = x0 + ks1; x1 = x1 + ks2 + np.uint32(1)
        x0, x1 = round4(x0, x1, rot_b)
        x0 = x0 + ks2; x1 = x1 + ks0 + np.uint32(2)
        x0, x1 = round4(x0, x1, rot_a)
        x0 = x0 + ks0; x1 = x1 + ks1 + np.uint32(3)
        x0, x1 = round4(x0, x1, rot_b)
        x0 = x0 + ks1; x1 = x1 + ks2 + np.uint32(4)
        x0, x1 = round4(x0, x1, rot_a)
        x0 = x0 + ks2; x1 = x1 + ks0 + np.uint32(5)
    return x0, x1


def _np_permutation(seed: int, n: int) -> np.ndarray:
    k0, k1 = np.uint32(seed >> 32), np.uint32(seed & 0xFFFFFFFF)
    x = np.arange(n, dtype=np.int32)
    iota_hi = np.zeros(n, dtype=np.uint32)
    iota_lo = np.arange(n, dtype=np.uint32)
    two_hi = np.zeros(2, dtype=np.uint32)
    two_lo = np.arange(2, dtype=np.uint32)
    num_rounds = int(np.ceil(3 * np.log(max(1, n)) / np.log(2**32 - 1)))
    for _ in range(num_rounds):
        y0, y1 = _threefry_block(k0, k1, two_hi, two_lo)
        (k0, k1), (s0, s1) = (y0[0], y1[0]), (y0[1], y1[1])
        b0, b1 = _threefry_block(s0, s1, iota_hi, iota_lo)
        x = x[np.argsort(b0 ^ b1, kind="stable")]
    return x


def _keep_indices(num_edges: int) -> np.ndarray:
    if num_edges not in _keep_cache:
        perm = _np_permutation(42, num_edges)
        num_drops = int(0.5 * num_edges)
        _keep_cache[num_edges] = perm[num_drops:].astype(np.int32)
    return _keep_cache[num_edges]


def _make_gather(E: int, K: int, D: int, chunk: int):
    per_w = K // _NW
    n_chunks = per_w // chunk
    mesh = plsc.VectorSubcoreMesh(core_axis_name="c", subcore_axis_name="s", num_cores=_NC, num_subcores=_NS)

    @functools.partial(
        pl.kernel,
        mesh=mesh,
        out_type=tuple(
            jax.ShapeDtypeStruct((K,), jnp.int32) for _ in range(2)
        ) + tuple(
            jax.ShapeDtypeStruct((K,), jnp.float32) for _ in range(D)
        ),
        scratch_types=[
            pltpu.VMEM((chunk,), jnp.int32),
            pltpu.VMEM((chunk,), jnp.int32),
            pltpu.VMEM((chunk,), jnp.int32),
        ] + [
            pltpu.VMEM((chunk,), jnp.float32) for _ in range(D)
        ] + [
            pltpu.SemaphoreType.DMA,
        ],
        compiler_params=pltpu.CompilerParams(use_tc_tiling_on_sc=False),
    )
    def gather_kernel(*refs):
        nt = 2 + D
        tables = refs[:nt]
        keep_lo = refs[nt]
        outs = refs[nt + 1:2 * nt + 1]
        idx_v = refs[2 * nt + 1]
        vals = refs[2 * nt + 2:3 * nt + 2]
        sem = refs[-1]
        wid = lax.axis_index("s") * _NC + lax.axis_index("c")
        w_base = wid * per_w

        def body(c, carry):
            base = w_base + c * chunk
            pltpu.sync_copy(keep_lo.at[pl.ds(base, chunk)], idx_v)
            cps = [pltpu.async_copy(t.at[idx_v], v, sem)
                   for t, v in zip(tables, vals)]
            for cp in cps:
                cp.wait()
            for v, o in zip(vals, outs):
                pltpu.sync_copy(v, o.at[pl.ds(base, chunk)])
            return carry

        lax.fori_loop(0, n_chunks, body, 0)

    return gather_kernel


def kernel(edge_index, edge_attr):
    E = edge_index.shape[1]
    D = edge_attr.shape[1]
    K = E - int(0.5 * E)
    keep = _keep_indices(E)
    keep_lo = jnp.asarray(keep)

    chunk = 5000
    assert K % (_NW * chunk) == 0

    attr_t = edge_attr.T
    tables = [edge_index[0], edge_index[1]] + [attr_t[j] for j in range(D)]
    gather_kernel = _make_gather(E, K, D, chunk)
    outs = gather_kernel(*tables, keep_lo)
    new_edge_index = jnp.stack([outs[0], outs[1]])
    new_edge_attr = jnp.stack(outs[2:], axis=1)
    return new_edge_index, new_edge_attr
